# baseline (speedup 1.0000x reference)
import jax
import jax.numpy as jnp
from jax.experimental import pallas as pl
from jax.experimental.pallas import tpu as pltpu

_Q_MIN = 0.01
_N = 100000
_LANES = 128
_ROWS = 832
_NPAD = _ROWS * _LANES
_CROWS = 32
_NCHUNK = _ROWS // _CROWS


def _tree(m, op):
    m = op(m[:416], m[416:])
    m = op(m[:208], m[208:])
    return op(m[:104], m[104:])


def _loss_kernel(beta_ref, pid_ref, x0_ref, x1_ref, x2_ref, out_ref,
                 q_ref, xx_ref, qa_s, a0_s, a1_s, a2_s, aa_s):
    beta = beta_ref[...]
    pid = pid_ref[...]
    x0 = x0_ref[...]
    x1 = x1_ref[...]
    x2 = x2_ref[...]

    at = 0.5 * jnp.log((1.0 + beta) / (1.0 - beta))
    q = at * at + _Q_MIN
    ridx = jax.lax.broadcasted_iota(jnp.int32, (_ROWS, _LANES), 0)
    cidx = jax.lax.broadcasted_iota(jnp.int32, (_ROWS, _LANES), 1)
    flat = ridx * _LANES + cidx
    q = jnp.where(flat < _N, q, 0.0)
    q_ref[...] = q
    xx_ref[...] = x0 * x0 + x1 * x1 + x2 * x2
    flat_f = flat.astype(jnp.float32)
    lane = jax.lax.broadcasted_iota(jnp.int32, (1, _LANES), 1)

    def phase1(p, carry):
        masked_q = jnp.where(pid == p, q, 0.0)
        qa = jnp.max(_tree(masked_q, jnp.maximum))
        mi = jnp.min(_tree(jnp.where(masked_q == qa, flat_f, 3.0e38),
                           jnp.minimum))
        mi_i = mi.astype(jnp.int32)
        r = mi_i >> 7
        c = mi_i & 127
        onlane = lane == c
        a0 = jnp.sum(jnp.where(onlane, x0_ref[pl.ds(r, 1), :], 0.0))
        a1 = jnp.sum(jnp.where(onlane, x1_ref[pl.ds(r, 1), :], 0.0))
        a2 = jnp.sum(jnp.where(onlane, x2_ref[pl.ds(r, 1), :], 0.0))
        qa_s[p] = qa
        a0_s[p] = a0
        a1_s[p] = a1
        a2_s[p] = a2
        aa_s[p] = a0 * a0 + a1 * a1 + a2 * a2
        return carry

    jax.lax.fori_loop(1, 50, phase1, jnp.int32(0))

    def phase2(ch, acc8):
        off = ch * _CROWS
        sl = pl.ds(off, _CROWS)
        x0c = x0_ref[sl, :]
        x1c = x1_ref[sl, :]
        x2c = x2_ref[sl, :]
        pidc = pid_ref[sl, :]
        qc = q_ref[sl, :]
        xxc = xx_ref[sl, :]

        def inner(p, acc_c):
            qa = qa_s[p]
            a0 = a0_s[p]
            a1 = a1_s[p]
            a2 = a2_s[p]
            aa = aa_s[p]
            t = x0c * a0 + x1c * a1 + x2c * a2
            dist2 = jnp.maximum((xxc - 2.0 * t) + aa, 0.0)
            norm = jnp.sqrt(dist2)
            rep10 = jnp.maximum(10.0 - 10.0 * norm, 0.0)
            val = jnp.where(pidc == p, dist2, rep10)
            return acc_c + qa * (qc * val)

        acc_c = jax.lax.fori_loop(
            1, 50, inner, jnp.zeros((_CROWS, _LANES), jnp.float32))
        return acc8 + (acc_c[:8] + acc_c[8:16] + acc_c[16:24] + acc_c[24:32])

    acc8 = jax.lax.fori_loop(0, _NCHUNK, phase2,
                             jnp.zeros((8, _LANES), jnp.float32))
    out_ref[0, 0] = jnp.sum(acc8) * (1.0 / _N)


def kernel(w, beta, x, y, particle_id):
    del w, y
    pid = particle_id.reshape(-1).astype(jnp.int32)
    pad = _NPAD - _N
    beta_p = jnp.pad(beta, (0, pad)).reshape(_ROWS, _LANES)
    pid_p = jnp.pad(pid, (0, pad)).reshape(_ROWS, _LANES)
    x_p = jnp.pad(x.astype(jnp.float32), ((0, pad), (0, 0)))
    x0 = x_p[:, 0].reshape(_ROWS, _LANES)
    x1 = x_p[:, 1].reshape(_ROWS, _LANES)
    x2 = x_p[:, 2].reshape(_ROWS, _LANES)

    out = pl.pallas_call(
        _loss_kernel,
        out_shape=jax.ShapeDtypeStruct((1, 1), jnp.float32),
        in_specs=[pl.BlockSpec((_ROWS, _LANES), lambda: (0, 0))] * 5,
        out_specs=pl.BlockSpec(memory_space=pltpu.SMEM),
        scratch_shapes=[
            pltpu.VMEM((_ROWS, _LANES), jnp.float32),
            pltpu.VMEM((_ROWS, _LANES), jnp.float32),
            pltpu.SMEM((64,), jnp.float32),
            pltpu.SMEM((64,), jnp.float32),
            pltpu.SMEM((64,), jnp.float32),
            pltpu.SMEM((64,), jnp.float32),
            pltpu.SMEM((64,), jnp.float32),
        ],
    )(beta_p, pid_p, x0, x1, x2)
    return out[0, 0]


# speedup vs baseline: 1.3137x; 1.3137x over previous
import jax
import jax.numpy as jnp
from jax.experimental import pallas as pl
from jax.experimental.pallas import tpu as pltpu

_Q_MIN = 0.01
_N = 100000
_LANES = 128
_ROWS = 832
_NPAD = _ROWS * _LANES
_CROWS = 32
_NCHUNK = _ROWS // _CROWS


def _tree(m, op):
    m = op(m[:416], m[416:])
    m = op(m[:208], m[208:])
    return op(m[:104], m[104:])


def _loss_kernel(beta_ref, pid_ref, x0_ref, x1_ref, x2_ref, out_ref,
                 q_ref, xx_ref, qa_s, a0_s, a1_s, a2_s, aa_s):
    beta = beta_ref[...]
    pid = pid_ref[...]
    x0 = x0_ref[...]
    x1 = x1_ref[...]
    x2 = x2_ref[...]

    at = 0.5 * jnp.log((1.0 + beta) / (1.0 - beta))
    q = at * at + _Q_MIN
    ridx = jax.lax.broadcasted_iota(jnp.int32, (_ROWS, _LANES), 0)
    cidx = jax.lax.broadcasted_iota(jnp.int32, (_ROWS, _LANES), 1)
    flat = ridx * _LANES + cidx
    q = jnp.where(flat < _N, q, 0.0)
    q_ref[...] = q
    xx_ref[...] = x0 * x0 + x1 * x1 + x2 * x2
    flat_f = flat.astype(jnp.float32)
    lane = jax.lax.broadcasted_iota(jnp.int32, (1, _LANES), 1)

    def phase1(p, carry):
        masked_q = jnp.where(pid == p, q, 0.0)
        qa = jnp.max(_tree(masked_q, jnp.maximum))
        mi = jnp.min(_tree(jnp.where(masked_q == qa, flat_f, 3.0e38),
                           jnp.minimum))
        mi_i = mi.astype(jnp.int32)
        r = mi_i >> 7
        c = mi_i & 127
        onlane = lane == c
        a0 = jnp.sum(jnp.where(onlane, x0_ref[pl.ds(r, 1), :], 0.0))
        a1 = jnp.sum(jnp.where(onlane, x1_ref[pl.ds(r, 1), :], 0.0))
        a2 = jnp.sum(jnp.where(onlane, x2_ref[pl.ds(r, 1), :], 0.0))
        qa_s[p] = qa
        a0_s[p] = a0
        a1_s[p] = a1
        a2_s[p] = a2
        aa_s[p] = a0 * a0 + a1 * a1 + a2 * a2
        return carry

    jax.lax.fori_loop(1, 50, phase1, jnp.int32(0))

    def phase2(ch, acc8):
        off = ch * _CROWS
        sl = pl.ds(off, _CROWS)
        x0c = x0_ref[sl, :]
        x1c = x1_ref[sl, :]
        x2c = x2_ref[sl, :]
        pidc = pid_ref[sl, :]
        qc = q_ref[sl, :]
        xxc = xx_ref[sl, :]

        acc_c = jnp.zeros((_CROWS, _LANES), jnp.float32)
        for p in range(1, 50):
            qa = qa_s[p]
            a0 = a0_s[p]
            a1 = a1_s[p]
            a2 = a2_s[p]
            aa = aa_s[p]
            t = x0c * a0 + x1c * a1 + x2c * a2
            dist2 = jnp.maximum((xxc - 2.0 * t) + aa, 0.0)
            norm = jnp.sqrt(dist2)
            rep10 = jnp.maximum(10.0 - 10.0 * norm, 0.0)
            val = jnp.where(pidc == p, dist2, rep10)
            acc_c = acc_c + qa * (qc * val)
        return acc8 + (acc_c[:8] + acc_c[8:16] + acc_c[16:24] + acc_c[24:32])

    acc8 = jax.lax.fori_loop(0, _NCHUNK, phase2,
                             jnp.zeros((8, _LANES), jnp.float32))
    out_ref[0, 0] = jnp.sum(acc8) * (1.0 / _N)


def kernel(w, beta, x, y, particle_id):
    del w, y
    pid = particle_id.reshape(-1).astype(jnp.int32)
    pad = _NPAD - _N
    beta_p = jnp.pad(beta, (0, pad)).reshape(_ROWS, _LANES)
    pid_p = jnp.pad(pid, (0, pad)).reshape(_ROWS, _LANES)
    x_p = jnp.pad(x.astype(jnp.float32), ((0, pad), (0, 0)))
    x0 = x_p[:, 0].reshape(_ROWS, _LANES)
    x1 = x_p[:, 1].reshape(_ROWS, _LANES)
    x2 = x_p[:, 2].reshape(_ROWS, _LANES)

    out = pl.pallas_call(
        _loss_kernel,
        out_shape=jax.ShapeDtypeStruct((1, 1), jnp.float32),
        in_specs=[pl.BlockSpec((_ROWS, _LANES), lambda: (0, 0))] * 5,
        out_specs=pl.BlockSpec(memory_space=pltpu.SMEM),
        scratch_shapes=[
            pltpu.VMEM((_ROWS, _LANES), jnp.float32),
            pltpu.VMEM((_ROWS, _LANES), jnp.float32),
            pltpu.SMEM((64,), jnp.float32),
            pltpu.SMEM((64,), jnp.float32),
            pltpu.SMEM((64,), jnp.float32),
            pltpu.SMEM((64,), jnp.float32),
            pltpu.SMEM((64,), jnp.float32),
        ],
    )(beta_p, pid_p, x0, x1, x2)
    return out[0, 0]


# speedup vs baseline: 1.4851x; 1.1305x over previous
import jax
import jax.numpy as jnp
from jax.experimental import pallas as pl
from jax.experimental.pallas import tpu as pltpu

_Q_MIN = 0.01
_N = 100000
_LANES = 128
_ROWS = 832
_NPAD = _ROWS * _LANES
_CROWS = 32
_NCHUNK = _ROWS // _CROWS


def _loss_kernel(beta_ref, pid_ref, x0_ref, x1_ref, x2_ref, out_ref,
                 q_ref, xx_ref, qa_s, a0_s, a1_s, a2_s, aa_s):
    beta = beta_ref[...]
    x0 = x0_ref[...]
    x1 = x1_ref[...]
    x2 = x2_ref[...]

    at = 0.5 * jnp.log((1.0 + beta) / (1.0 - beta))
    q = at * at + _Q_MIN
    ridx = jax.lax.broadcasted_iota(jnp.int32, (_ROWS, _LANES), 0)
    cidx = jax.lax.broadcasted_iota(jnp.int32, (_ROWS, _LANES), 1)
    q = jnp.where(ridx * _LANES + cidx < _N, q, 0.0)
    q_ref[...] = q
    xx_ref[...] = x0 * x0 + x1 * x1 + x2 * x2
    lane = jax.lax.broadcasted_iota(jnp.int32, (1, _LANES), 1)
    rowpids = jax.lax.broadcasted_iota(jnp.int32, (64, _LANES), 0)

    def p1_chunk(ch, carry):
        tabm, tabi = carry
        off = ch * _CROWS
        for rr in range(_CROWS):
            r = off + rr
            q_r = q_ref[pl.ds(r, 1), :]
            pid_r = pid_ref[pl.ds(r, 1), :]
            upd = jnp.logical_and(pid_r == rowpids, q_r > tabm)
            tabm = jnp.where(upd, q_r, tabm)
            tabi = jnp.where(upd, r * _LANES + lane, tabi)
        return tabm, tabi

    tabm, tabi = jax.lax.fori_loop(
        0, _NCHUNK, p1_chunk,
        (jnp.zeros((64, _LANES), jnp.float32),
         jnp.zeros((64, _LANES), jnp.int32)))

    for p in range(1, 50):
        rowm = tabm[p:p + 1]
        rowi = tabi[p:p + 1]
        qa = jnp.max(rowm)
        mi = jnp.min(jnp.where(rowm == qa, rowi, jnp.int32(1 << 30)))
        r = mi >> 7
        c = mi & 127
        onlane = lane == c
        a0 = jnp.sum(jnp.where(onlane, x0_ref[pl.ds(r, 1), :], 0.0))
        a1 = jnp.sum(jnp.where(onlane, x1_ref[pl.ds(r, 1), :], 0.0))
        a2 = jnp.sum(jnp.where(onlane, x2_ref[pl.ds(r, 1), :], 0.0))
        qa_s[p] = qa
        a0_s[p] = a0
        a1_s[p] = a1
        a2_s[p] = a2
        aa_s[p] = a0 * a0 + a1 * a1 + a2 * a2

    def phase2(ch, acc8):
        off = ch * _CROWS
        sl = pl.ds(off, _CROWS)
        x0c = x0_ref[sl, :]
        x1c = x1_ref[sl, :]
        x2c = x2_ref[sl, :]
        pidc = pid_ref[sl, :]
        qc = q_ref[sl, :]
        xxc = xx_ref[sl, :]

        acc_c = jnp.zeros((_CROWS, _LANES), jnp.float32)
        for p in range(1, 50):
            qa = qa_s[p]
            a0 = a0_s[p]
            a1 = a1_s[p]
            a2 = a2_s[p]
            aa = aa_s[p]
            t = x0c * a0 + x1c * a1 + x2c * a2
            dist2 = jnp.maximum((xxc - 2.0 * t) + aa, 0.0)
            norm = jnp.sqrt(dist2)
            rep10 = jnp.maximum(10.0 - 10.0 * norm, 0.0)
            val = jnp.where(pidc == p, dist2, rep10)
            acc_c = acc_c + qa * val
        acc_c = qc * acc_c
        return acc8 + (acc_c[:8] + acc_c[8:16] + acc_c[16:24] + acc_c[24:32])

    acc8 = jax.lax.fori_loop(0, _NCHUNK, phase2,
                             jnp.zeros((8, _LANES), jnp.float32))
    out_ref[0, 0] = jnp.sum(acc8) * (1.0 / _N)


def kernel(w, beta, x, y, particle_id):
    del w, y
    pid = particle_id.reshape(-1).astype(jnp.int32)
    pad = _NPAD - _N
    beta_p = jnp.pad(beta, (0, pad)).reshape(_ROWS, _LANES)
    pid_p = jnp.pad(pid, (0, pad)).reshape(_ROWS, _LANES)
    x_p = jnp.pad(x.astype(jnp.float32), ((0, pad), (0, 0)))
    x0 = x_p[:, 0].reshape(_ROWS, _LANES)
    x1 = x_p[:, 1].reshape(_ROWS, _LANES)
    x2 = x_p[:, 2].reshape(_ROWS, _LANES)

    out = pl.pallas_call(
        _loss_kernel,
        out_shape=jax.ShapeDtypeStruct((1, 1), jnp.float32),
        in_specs=[pl.BlockSpec((_ROWS, _LANES), lambda: (0, 0))] * 5,
        out_specs=pl.BlockSpec(memory_space=pltpu.SMEM),
        scratch_shapes=[
            pltpu.VMEM((_ROWS, _LANES), jnp.float32),
            pltpu.VMEM((_ROWS, _LANES), jnp.float32),
            pltpu.SMEM((64,), jnp.float32),
            pltpu.SMEM((64,), jnp.float32),
            pltpu.SMEM((64,), jnp.float32),
            pltpu.SMEM((64,), jnp.float32),
            pltpu.SMEM((64,), jnp.float32),
        ],
    )(beta_p, pid_p, x0, x1, x2)
    return out[0, 0]


# speedup vs baseline: 2.2761x; 1.5326x over previous
import jax
import jax.numpy as jnp
from jax.experimental import pallas as pl
from jax.experimental.pallas import tpu as pltpu

_Q_MIN = 0.01
_N = 100000
_LANES = 128
_ROWS = 832
_NPAD = _ROWS * _LANES
_CROWS = 32
_NCHUNK = _ROWS // _CROWS


def _loss_kernel(beta_ref, pid_ref, x0_ref, x1_ref, x2_ref, out_ref,
                 q_ref, xx_ref, r0_ref, r1_ref, r2_ref,
                 qa_ref, a0_ref, a1_ref, a2_ref, aa_ref):
    beta = beta_ref[...]
    x0 = x0_ref[...]
    x1 = x1_ref[...]
    x2 = x2_ref[...]

    at = 0.5 * jnp.log((1.0 + beta) / (1.0 - beta))
    q = at * at + _Q_MIN
    ridx = jax.lax.broadcasted_iota(jnp.int32, (_ROWS, _LANES), 0)
    cidx = jax.lax.broadcasted_iota(jnp.int32, (_ROWS, _LANES), 1)
    q = jnp.where(ridx * _LANES + cidx < _N, q, 0.0)
    q_ref[...] = q
    xx_ref[...] = x0 * x0 + x1 * x1 + x2 * x2
    lane = jax.lax.broadcasted_iota(jnp.int32, (1, _LANES), 1)
    rowpids = jax.lax.broadcasted_iota(jnp.int32, (64, _LANES), 0)

    def p1_chunk(ch, carry):
        tabm, tabi = carry
        off = ch * _CROWS
        for rr in range(_CROWS):
            r = off + rr
            q_r = q_ref[pl.ds(r, 1), :]
            pid_r = pid_ref[pl.ds(r, 1), :]
            upd = jnp.logical_and(pid_r == rowpids, q_r > tabm)
            tabm = jnp.where(upd, q_r, tabm)
            tabi = jnp.where(upd, r * _LANES + lane, tabi)
        return tabm, tabi

    tabm, tabi = jax.lax.fori_loop(
        0, _NCHUNK, p1_chunk,
        (jnp.zeros((64, _LANES), jnp.float32),
         jnp.zeros((64, _LANES), jnp.int32)))

    qa64 = jnp.max(tabm, axis=1, keepdims=True)
    mi64 = jnp.min(jnp.where(tabm == qa64, tabi, jnp.int32(1 << 30)),
                   axis=1, keepdims=True)
    r64 = mi64 >> 7
    c64 = mi64 & 127

    for p in range(1, 50):
        r = r64[p, 0]
        r0_ref[p:p + 1, :] = x0_ref[pl.ds(r, 1), :]
        r1_ref[p:p + 1, :] = x1_ref[pl.ds(r, 1), :]
        r2_ref[p:p + 1, :] = x2_ref[pl.ds(r, 1), :]

    onlane = lane == c64
    a0_64 = jnp.sum(jnp.where(onlane, r0_ref[...], 0.0), axis=1,
                    keepdims=True)
    a1_64 = jnp.sum(jnp.where(onlane, r1_ref[...], 0.0), axis=1,
                    keepdims=True)
    a2_64 = jnp.sum(jnp.where(onlane, r2_ref[...], 0.0), axis=1,
                    keepdims=True)
    qa_ref[...] = jnp.broadcast_to(qa64, (64, _LANES))
    a0_ref[...] = jnp.broadcast_to(a0_64, (64, _LANES))
    a1_ref[...] = jnp.broadcast_to(a1_64, (64, _LANES))
    a2_ref[...] = jnp.broadcast_to(a2_64, (64, _LANES))
    aa_ref[...] = jnp.broadcast_to(
        a0_64 * a0_64 + a1_64 * a1_64 + a2_64 * a2_64, (64, _LANES))

    def phase2(ch, acc8):
        off = ch * _CROWS
        sl = pl.ds(off, _CROWS)
        x0c = x0_ref[sl, :]
        x1c = x1_ref[sl, :]
        x2c = x2_ref[sl, :]
        pidc = pid_ref[sl, :]
        qc = q_ref[sl, :]
        xxc = xx_ref[sl, :]

        acc_c = jnp.zeros((_CROWS, _LANES), jnp.float32)
        for p in range(1, 50):
            qa = qa_ref[p:p + 1, :]
            a0 = a0_ref[p:p + 1, :]
            a1 = a1_ref[p:p + 1, :]
            a2 = a2_ref[p:p + 1, :]
            aa = aa_ref[p:p + 1, :]
            t = x0c * a0 + x1c * a1 + x2c * a2
            dist2 = jnp.maximum((xxc - 2.0 * t) + aa, 0.0)
            norm = jnp.sqrt(dist2)
            rep10 = jnp.maximum(10.0 - 10.0 * norm, 0.0)
            val = jnp.where(pidc == p, dist2, rep10)
            acc_c = acc_c + qa * val
        acc_c = qc * acc_c
        return acc8 + (acc_c[:8] + acc_c[8:16] + acc_c[16:24] + acc_c[24:32])

    acc8 = jax.lax.fori_loop(0, _NCHUNK, phase2,
                             jnp.zeros((8, _LANES), jnp.float32))
    out_ref[0, 0] = jnp.sum(acc8) * (1.0 / _N)


def kernel(w, beta, x, y, particle_id):
    del w, y
    pid = particle_id.reshape(-1).astype(jnp.int32)
    pad = _NPAD - _N
    beta_p = jnp.pad(beta, (0, pad)).reshape(_ROWS, _LANES)
    pid_p = jnp.pad(pid, (0, pad)).reshape(_ROWS, _LANES)
    x_p = jnp.pad(x.astype(jnp.float32), ((0, pad), (0, 0)))
    x0 = x_p[:, 0].reshape(_ROWS, _LANES)
    x1 = x_p[:, 1].reshape(_ROWS, _LANES)
    x2 = x_p[:, 2].reshape(_ROWS, _LANES)

    out = pl.pallas_call(
        _loss_kernel,
        out_shape=jax.ShapeDtypeStruct((1, 1), jnp.float32),
        in_specs=[pl.BlockSpec((_ROWS, _LANES), lambda: (0, 0))] * 5,
        out_specs=pl.BlockSpec(memory_space=pltpu.SMEM),
        scratch_shapes=[
            pltpu.VMEM((_ROWS, _LANES), jnp.float32),
            pltpu.VMEM((_ROWS, _LANES), jnp.float32),
            pltpu.VMEM((64, _LANES), jnp.float32),
            pltpu.VMEM((64, _LANES), jnp.float32),
            pltpu.VMEM((64, _LANES), jnp.float32),
            pltpu.VMEM((64, _LANES), jnp.float32),
            pltpu.VMEM((64, _LANES), jnp.float32),
            pltpu.VMEM((64, _LANES), jnp.float32),
            pltpu.VMEM((64, _LANES), jnp.float32),
            pltpu.VMEM((64, _LANES), jnp.float32),
        ],
    )(beta_p, pid_p, x0, x1, x2)
    return out[0, 0]
